# Initial kernel scaffold; baseline (speedup 1.0000x reference)
import jax
import jax.numpy as jnp
from jax.experimental import pallas as pl


def kernel(x, edge_index, edge_type, edge_norm, basis, root, bias):
    raise NotImplementedError("write your pallas kernel here")


# speedup vs baseline: 10.4755x; 10.4755x over previous
import functools

import jax
import jax.numpy as jnp
from jax import lax
from jax.experimental import pallas as pl
from jax.experimental.pallas import tpu as pltpu
from jax.experimental.pallas import tpu_sc as plsc

N = 10000
E = 320000
D = 128
R = 8

NC = 2
NS = 16
L = 16
NW = NC * NS
EPW = E // NW
K = 80
NCHUNK = EPW // K
RPT = 624
REXTRA = N - NS * RPT
BN = 1000


def _xw_body(x_ref, basis_ref, out_ref):
    xb = x_ref[...]
    for r in range(R):
        out_ref[r] = jnp.dot(xb, basis_ref[r],
                             preferred_element_type=jnp.float32)


def _combine_body(p_ref, x_ref, root_ref, bias_ref, out_ref):
    out_ref[...] = (p_ref[0] + p_ref[1] + bias_ref[...]
                    + jnp.dot(x_ref[...], root_ref[...],
                              preferred_element_type=jnp.float32))


def _sc_edge_kernel(xw_hbm, src_hbm, dst_hbm, typ_hbm, nrm_hbm, part_hbm,
                    srcv, typv, dstv, nrmv, gidxv, rowsv, acc, gsem):
    cid = lax.axis_index("c")
    sid = lax.axis_index("s")
    wid = sid * NC + cid

    def zrow(i, carry):
        for g in range(D // L):
            rowsv[i, pl.ds(g * L, L)] = jnp.zeros((L,), jnp.float32)
        return carry
    lax.fori_loop(0, K, zrow, 0)
    r0 = sid * RPT
    nfull = RPT // K
    rem = RPT - nfull * K

    def zacc(j, carry):
        pltpu.sync_copy(rowsv, acc.at[pl.ds(r0 + j * K, K)])
        return carry
    lax.fori_loop(0, nfull, zacc, 0)
    if rem:
        pltpu.sync_copy(rowsv.at[pl.ds(0, rem)],
                        acc.at[pl.ds(r0 + nfull * K, rem)])

    @pl.when(sid == NS - 1)
    def _zero_tail():
        pltpu.sync_copy(rowsv.at[pl.ds(0, REXTRA)],
                        acc.at[pl.ds(NS * RPT, REXTRA)])
    plsc.subcore_barrier()

    def chunk_body(c, carry):
        base = pl.multiple_of(wid * EPW + c * K, 8)
        pltpu.sync_copy(src_hbm.at[pl.ds(base, K)], srcv)
        pltpu.sync_copy(typ_hbm.at[pl.ds(base, K)], typv)
        pltpu.sync_copy(dst_hbm.at[pl.ds(base, K)], dstv)
        pltpu.sync_copy(nrm_hbm.at[pl.ds(base, K)], nrmv)
        for g in range(K // L):
            sl = pl.ds(g * L, L)
            gidxv[sl] = typv[sl] * N + srcv[sl]
        pltpu.async_copy(xw_hbm.at[gidxv], rowsv, gsem).wait()

        def scale_body(e, carry2):
            nb = plsc.load_gather(nrmv, (jnp.full((L,), e, jnp.int32),))
            for g in range(D // L):
                sl = pl.ds(g * L, L)
                rowsv[e, sl] = rowsv[e, sl] * nb
            return carry2
        lax.fori_loop(0, K, scale_body, 0)
        pltpu.sync_copy(rowsv, acc.at[dstv], add=True)
        return carry
    lax.fori_loop(0, NCHUNK, chunk_body, 0)

    plsc.subcore_barrier()
    pltpu.sync_copy(acc.at[pl.ds(r0, RPT)], part_hbm.at[cid, pl.ds(r0, RPT)])

    @pl.when(sid == NS - 1)
    def _write_tail():
        pltpu.sync_copy(acc.at[pl.ds(NS * RPT, REXTRA)],
                        part_hbm.at[cid, pl.ds(NS * RPT, REXTRA)])


_sc_edge = functools.partial(
    pl.kernel,
    out_type=jax.ShapeDtypeStruct((NC, N, D), jnp.float32),
    mesh=plsc.VectorSubcoreMesh(core_axis_name="c", subcore_axis_name="s"),
    compiler_params=pltpu.CompilerParams(needs_layout_passes=False),
    scratch_types=[
        pltpu.VMEM((K,), jnp.int32),
        pltpu.VMEM((K,), jnp.int32),
        pltpu.VMEM((K,), jnp.int32),
        pltpu.VMEM((K,), jnp.float32),
        pltpu.VMEM((K,), jnp.int32),
        pltpu.VMEM((K, D), jnp.float32),
        pltpu.VMEM_SHARED((N, D), jnp.float32),
        pltpu.SemaphoreType.DMA,
    ],
)(_sc_edge_kernel)


def kernel(x, edge_index, edge_type, edge_norm, basis, root, bias):
    src = edge_index[0]
    dst = edge_index[1]

    xw = pl.pallas_call(
        _xw_body,
        grid=(N // BN,),
        in_specs=[
            pl.BlockSpec((BN, D), lambda i: (i, 0)),
            pl.BlockSpec((R, D, D), lambda i: (0, 0, 0)),
        ],
        out_specs=pl.BlockSpec((R, BN, D), lambda i: (0, i, 0)),
        out_shape=jax.ShapeDtypeStruct((R, N, D), jnp.float32),
    )(x, basis)
    xw_flat = xw.reshape(R * N, D)

    partial = _sc_edge(xw_flat, src, dst, edge_type, edge_norm)

    out = pl.pallas_call(
        _combine_body,
        grid=(N // BN,),
        in_specs=[
            pl.BlockSpec((NC, BN, D), lambda i: (0, i, 0)),
            pl.BlockSpec((BN, D), lambda i: (i, 0)),
            pl.BlockSpec((D, D), lambda i: (0, 0)),
            pl.BlockSpec((1, D), lambda i: (0, 0)),
        ],
        out_specs=pl.BlockSpec((BN, D), lambda i: (i, 0)),
        out_shape=jax.ShapeDtypeStruct((N, D), jnp.float32),
    )(partial, x, root, bias.reshape(1, D))
    return out


# speedup vs baseline: 22.6220x; 2.1595x over previous
import functools

import jax
import jax.numpy as jnp
from jax import lax
from jax.experimental import pallas as pl
from jax.experimental.pallas import tpu as pltpu
from jax.experimental.pallas import tpu_sc as plsc

N = 10000
E = 320000
D = 128
R = 8

NC = 2
NS = 16
L = 16
NW = NC * NS
EPW = E // NW
K = 128
NCHUNK = EPW // K
KTAIL = EPW - NCHUNK * K
RPT = 624
REXTRA = N - NS * RPT
BN = 1000


def _xw_body(x_ref, basis_ref, out_ref):
    xb = x_ref[...]
    for r in range(R):
        out_ref[r] = jnp.dot(xb, basis_ref[r],
                             preferred_element_type=jnp.float32)


def _combine_body(p_ref, x_ref, root_ref, bias_ref, out_ref):
    out_ref[...] = (p_ref[0] + p_ref[1] + bias_ref[...]
                    + jnp.dot(x_ref[...], root_ref[...],
                              preferred_element_type=jnp.float32))


def _sc_edge_kernel(xw_hbm, src_hbm, dst_hbm, typ_hbm, nrm_hbm, part_hbm,
                    srcv, typv, dstv, nrmv, gidxv, rowsv,
                    dstt, gidxt, acc, msem0, msem1, gsem0, gsem1):
    cid = lax.axis_index("c")
    sid = lax.axis_index("s")
    wid = sid * NC + cid
    msem = (msem0, msem1)
    gsem = (gsem0, gsem1)

    def zrow(i, carry):
        for g in range(D // L):
            rowsv[0, i, pl.ds(g * L, L)] = jnp.zeros((L,), jnp.float32)
        return carry
    lax.fori_loop(0, K, zrow, 0)
    r0 = sid * RPT
    nfull = RPT // K
    rem = RPT - nfull * K

    def zacc(j, carry):
        pltpu.sync_copy(rowsv.at[0], acc.at[pl.ds(r0 + j * K, K)])
        return carry
    lax.fori_loop(0, nfull, zacc, 0)
    if rem:
        pltpu.sync_copy(rowsv.at[0, pl.ds(0, rem)],
                        acc.at[pl.ds(r0 + nfull * K, rem)])

    @pl.when(sid == NS - 1)
    def _zero_tail():
        pltpu.sync_copy(rowsv.at[0, pl.ds(0, REXTRA)],
                        acc.at[pl.ds(NS * RPT, REXTRA)])
    plsc.subcore_barrier()

    def m_start(c, b):
        base = pl.multiple_of(wid * EPW + c * K, 8)
        pltpu.async_copy(src_hbm.at[pl.ds(base, K)], srcv.at[b], msem[b])
        pltpu.async_copy(typ_hbm.at[pl.ds(base, K)], typv.at[b], msem[b])
        pltpu.async_copy(dst_hbm.at[pl.ds(base, K)], dstv.at[b], msem[b])
        pltpu.async_copy(nrm_hbm.at[pl.ds(base, K)], nrmv.at[b], msem[b])

    def m_wait(b):
        pltpu.make_async_copy(src_hbm.at[pl.ds(0, K)], srcv.at[b], msem[b]).wait()
        pltpu.make_async_copy(typ_hbm.at[pl.ds(0, K)], typv.at[b], msem[b]).wait()
        pltpu.make_async_copy(dst_hbm.at[pl.ds(0, K)], dstv.at[b], msem[b]).wait()
        pltpu.make_async_copy(nrm_hbm.at[pl.ds(0, K)], nrmv.at[b], msem[b]).wait()

    def gidx_compute(b):
        for g in range(K // L):
            sl = pl.ds(g * L, L)
            gidxv[b, sl] = typv[b, sl] * N + srcv[b, sl]

    def g_start(b):
        pltpu.async_copy(xw_hbm.at[gidxv.at[b]], rowsv.at[b], gsem[b])

    def g_wait(b):
        pltpu.make_async_copy(xw_hbm.at[gidxv.at[b]], rowsv.at[b],
                              gsem[b]).wait()

    def scale(b):
        def scale_body(e4, carry2):
            for j in range(4):
                e = e4 * 4 + j
                nb = plsc.load_gather(nrmv.at[b],
                                      (jnp.full((L,), e, jnp.int32),))
                for g in range(D // L):
                    sl = pl.ds(g * L, L)
                    rowsv[b, e, sl] = rowsv[b, e, sl] * nb
            return carry2
        lax.fori_loop(0, K // 4, scale_body, 0)

    def step(c, b):
        nb = 1 - b

        @pl.when(c + 1 < NCHUNK)
        def _advance():
            m_wait(nb)
            gidx_compute(nb)
            g_start(nb)
        g_wait(b)
        scale(b)
        pltpu.sync_copy(rowsv.at[b], acc.at[dstv.at[b]], add=True)

        @pl.when(c + 2 < NCHUNK)
        def _prefetch():
            m_start(c + 2, b)

    m_start(0, 0)
    m_wait(0)
    gidx_compute(0)
    g_start(0)
    m_start(1, 1)

    def outer(i, carry):
        step(i * 2, 0)
        step(i * 2 + 1, 1)
        return carry
    lax.fori_loop(0, NCHUNK // 2, outer, 0)

    tbase = pl.multiple_of(wid * EPW + NCHUNK * K, 8)
    pltpu.sync_copy(src_hbm.at[pl.ds(tbase, KTAIL)],
                    srcv.at[0, pl.ds(0, KTAIL)])
    pltpu.sync_copy(typ_hbm.at[pl.ds(tbase, KTAIL)],
                    typv.at[0, pl.ds(0, KTAIL)])
    pltpu.sync_copy(dst_hbm.at[pl.ds(tbase, KTAIL)], dstt)
    pltpu.sync_copy(nrm_hbm.at[pl.ds(tbase, KTAIL)],
                    nrmv.at[0, pl.ds(0, KTAIL)])
    gidxt[...] = typv[0, pl.ds(0, L)] * N + srcv[0, pl.ds(0, L)]
    pltpu.async_copy(xw_hbm.at[gidxt], rowsv.at[0, pl.ds(0, KTAIL)],
                     gsem0).wait()

    def tail_scale(e, carry2):
        nb = plsc.load_gather(nrmv.at[0], (jnp.full((L,), e, jnp.int32),))
        for g in range(D // L):
            sl = pl.ds(g * L, L)
            rowsv[0, e, sl] = rowsv[0, e, sl] * nb
        return carry2
    lax.fori_loop(0, KTAIL, tail_scale, 0)
    pltpu.sync_copy(rowsv.at[0, pl.ds(0, KTAIL)], acc.at[dstt], add=True)

    plsc.subcore_barrier()
    pltpu.sync_copy(acc.at[pl.ds(r0, RPT)], part_hbm.at[cid, pl.ds(r0, RPT)])

    @pl.when(sid == NS - 1)
    def _write_tail():
        pltpu.sync_copy(acc.at[pl.ds(NS * RPT, REXTRA)],
                        part_hbm.at[cid, pl.ds(NS * RPT, REXTRA)])


_sc_edge = functools.partial(
    pl.kernel,
    out_type=jax.ShapeDtypeStruct((NC, N, D), jnp.float32),
    mesh=plsc.VectorSubcoreMesh(core_axis_name="c", subcore_axis_name="s"),
    compiler_params=pltpu.CompilerParams(needs_layout_passes=False),
    scratch_types=[
        pltpu.VMEM((2, K), jnp.int32),
        pltpu.VMEM((2, K), jnp.int32),
        pltpu.VMEM((2, K), jnp.int32),
        pltpu.VMEM((2, K), jnp.float32),
        pltpu.VMEM((2, K), jnp.int32),
        pltpu.VMEM((2, K, D), jnp.float32),
        pltpu.VMEM((KTAIL,), jnp.int32),
        pltpu.VMEM((L,), jnp.int32),
        pltpu.VMEM_SHARED((N, D), jnp.float32),
        pltpu.SemaphoreType.DMA,
        pltpu.SemaphoreType.DMA,
        pltpu.SemaphoreType.DMA,
        pltpu.SemaphoreType.DMA,
    ],
)(_sc_edge_kernel)


def kernel(x, edge_index, edge_type, edge_norm, basis, root, bias):
    src = edge_index[0]
    dst = edge_index[1]

    xw = pl.pallas_call(
        _xw_body,
        grid=(N // BN,),
        in_specs=[
            pl.BlockSpec((BN, D), lambda i: (i, 0)),
            pl.BlockSpec((R, D, D), lambda i: (0, 0, 0)),
        ],
        out_specs=pl.BlockSpec((R, BN, D), lambda i: (0, i, 0)),
        out_shape=jax.ShapeDtypeStruct((R, N, D), jnp.float32),
    )(x, basis)
    xw_flat = xw.reshape(R * N, D)

    partial = _sc_edge(xw_flat, src, dst, edge_type, edge_norm)

    out = pl.pallas_call(
        _combine_body,
        grid=(N // BN,),
        in_specs=[
            pl.BlockSpec((NC, BN, D), lambda i: (0, i, 0)),
            pl.BlockSpec((BN, D), lambda i: (i, 0)),
            pl.BlockSpec((D, D), lambda i: (0, 0)),
            pl.BlockSpec((1, D), lambda i: (0, 0)),
        ],
        out_specs=pl.BlockSpec((BN, D), lambda i: (i, 0)),
        out_shape=jax.ShapeDtypeStruct((N, D), jnp.float32),
    )(partial, x, root, bias.reshape(1, D))
    return out


# speedup vs baseline: 30.9320x; 1.3673x over previous
import functools

import jax
import jax.numpy as jnp
from jax import lax
from jax.experimental import pallas as pl
from jax.experimental.pallas import tpu as pltpu
from jax.experimental.pallas import tpu_sc as plsc

N = 10000
E = 320000
D = 128
R = 8

NC = 2
NS = 16
L = 16
NW = NC * NS
EPW = E // NW
K = 128
NCHUNK = EPW // K
KTAIL = EPW - NCHUNK * K
RPT = 624
REXTRA = N - NS * RPT
BN = 1000


def _xw_body(x_ref, basis_ref, out_ref):
    xb = x_ref[...]
    for r in range(R):
        out_ref[r] = jnp.dot(xb, basis_ref[r],
                             preferred_element_type=jnp.float32)


def _combine_body(p_ref, x_ref, root_ref, bias_ref, out_ref):
    out_ref[...] = (p_ref[0] + p_ref[1] + bias_ref[...]
                    + jnp.dot(x_ref[...], root_ref[...],
                              preferred_element_type=jnp.float32))


def _sc_edge_kernel(xw_hbm, src_hbm, dst_hbm, typ_hbm, nrm_hbm, part_hbm,
                    srcv, typv, dstv, nrmv, gidxv, rowsv, dsts,
                    dstt, gidxt, acc, msem0, msem1, gsem0, gsem1,
                    ssem0, ssem1):
    cid = lax.axis_index("c")
    sid = lax.axis_index("s")
    wid = sid * NC + cid
    msem = (msem0, msem1)
    gsem = (gsem0, gsem1)
    ssem = (ssem0, ssem1)

    def zrow(i, carry):
        for g in range(D // L):
            rowsv[0, i, pl.ds(g * L, L)] = jnp.zeros((L,), jnp.float32)
        return carry
    lax.fori_loop(0, K, zrow, 0)
    r0 = sid * RPT
    nfull = RPT // K
    rem = RPT - nfull * K

    def zacc(j, carry):
        pltpu.sync_copy(rowsv.at[0], acc.at[pl.ds(r0 + j * K, K)])
        return carry
    lax.fori_loop(0, nfull, zacc, 0)
    if rem:
        pltpu.sync_copy(rowsv.at[0, pl.ds(0, rem)],
                        acc.at[pl.ds(r0 + nfull * K, rem)])

    @pl.when(sid == NS - 1)
    def _zero_tail():
        pltpu.sync_copy(rowsv.at[0, pl.ds(0, REXTRA)],
                        acc.at[pl.ds(NS * RPT, REXTRA)])
    plsc.subcore_barrier()

    def m_start(c, b):
        base = pl.multiple_of(wid * EPW + c * K, 8)
        pltpu.async_copy(src_hbm.at[pl.ds(base, K)], srcv.at[b], msem[b])
        pltpu.async_copy(typ_hbm.at[pl.ds(base, K)], typv.at[b], msem[b])
        pltpu.async_copy(dst_hbm.at[pl.ds(base, K)], dstv.at[b], msem[b])
        pltpu.async_copy(nrm_hbm.at[pl.ds(base, K)], nrmv.at[b], msem[b])

    def m_wait(b):
        pltpu.make_async_copy(src_hbm.at[pl.ds(0, K)], srcv.at[b], msem[b]).wait()
        pltpu.make_async_copy(typ_hbm.at[pl.ds(0, K)], typv.at[b], msem[b]).wait()
        pltpu.make_async_copy(dst_hbm.at[pl.ds(0, K)], dstv.at[b], msem[b]).wait()
        pltpu.make_async_copy(nrm_hbm.at[pl.ds(0, K)], nrmv.at[b], msem[b]).wait()

    def gidx_compute(b):
        for g in range(K // L):
            sl = pl.ds(g * L, L)
            gidxv[b, sl] = typv[b, sl] * N + srcv[b, sl]

    def g_start(b):
        pltpu.async_copy(xw_hbm.at[gidxv.at[b]], rowsv.at[b], gsem[b])

    def g_wait(b):
        pltpu.make_async_copy(xw_hbm.at[gidxv.at[b]], rowsv.at[b],
                              gsem[b]).wait()

    def scale(b):
        @plsc.parallel_loop(0, K, 1, unroll=8)
        def _body(e):
            nb = plsc.load_gather(nrmv.at[b],
                                  (jnp.full((L,), e, jnp.int32),))
            for g in range(D // L):
                sl = pl.ds(g * L, L)
                rowsv[b, e, sl] = rowsv[b, e, sl] * nb

    def s_start(b):
        pltpu.async_copy(rowsv.at[b], acc.at[dsts.at[b]], ssem[b], add=True)

    def s_wait(b):
        pltpu.make_async_copy(rowsv.at[b], acc.at[dsts.at[b]],
                              ssem[b]).wait()

    def step(c, b):
        nb = 1 - b

        @pl.when(c + 1 < NCHUNK)
        def _advance():
            m_wait(nb)
            gidx_compute(nb)

            @pl.when(c >= 1)
            def _drain_prev_scatter():
                s_wait(nb)
            g_start(nb)
        g_wait(b)
        scale(b)
        for g in range(K // L):
            sl = pl.ds(g * L, L)
            dsts[b, sl] = dstv[b, sl]
        s_start(b)

        @pl.when(c + 2 < NCHUNK)
        def _prefetch():
            m_start(c + 2, b)

    m_start(0, 0)
    m_wait(0)
    gidx_compute(0)
    g_start(0)
    m_start(1, 1)

    def outer(i, carry):
        step(i * 2, 0)
        step(i * 2 + 1, 1)
        return carry
    lax.fori_loop(0, NCHUNK // 2, outer, 0)
    s_wait(0)
    s_wait(1)

    tbase = pl.multiple_of(wid * EPW + NCHUNK * K, 8)
    pltpu.sync_copy(src_hbm.at[pl.ds(tbase, KTAIL)],
                    srcv.at[0, pl.ds(0, KTAIL)])
    pltpu.sync_copy(typ_hbm.at[pl.ds(tbase, KTAIL)],
                    typv.at[0, pl.ds(0, KTAIL)])
    pltpu.sync_copy(dst_hbm.at[pl.ds(tbase, KTAIL)], dstt)
    pltpu.sync_copy(nrm_hbm.at[pl.ds(tbase, KTAIL)],
                    nrmv.at[0, pl.ds(0, KTAIL)])
    gidxt[...] = typv[0, pl.ds(0, L)] * N + srcv[0, pl.ds(0, L)]
    pltpu.async_copy(xw_hbm.at[gidxt], rowsv.at[0, pl.ds(0, KTAIL)],
                     gsem0).wait()

    def tail_scale(e, carry2):
        nb = plsc.load_gather(nrmv.at[0], (jnp.full((L,), e, jnp.int32),))
        for g in range(D // L):
            sl = pl.ds(g * L, L)
            rowsv[0, e, sl] = rowsv[0, e, sl] * nb
        return carry2
    lax.fori_loop(0, KTAIL, tail_scale, 0)
    pltpu.sync_copy(rowsv.at[0, pl.ds(0, KTAIL)], acc.at[dstt], add=True)

    plsc.subcore_barrier()
    pltpu.sync_copy(acc.at[pl.ds(r0, RPT)], part_hbm.at[cid, pl.ds(r0, RPT)])

    @pl.when(sid == NS - 1)
    def _write_tail():
        pltpu.sync_copy(acc.at[pl.ds(NS * RPT, REXTRA)],
                        part_hbm.at[cid, pl.ds(NS * RPT, REXTRA)])


_sc_edge = functools.partial(
    pl.kernel,
    out_type=jax.ShapeDtypeStruct((NC, N, D), jnp.float32),
    mesh=plsc.VectorSubcoreMesh(core_axis_name="c", subcore_axis_name="s"),
    compiler_params=pltpu.CompilerParams(needs_layout_passes=False),
    scratch_types=[
        pltpu.VMEM((2, K), jnp.int32),
        pltpu.VMEM((2, K), jnp.int32),
        pltpu.VMEM((2, K), jnp.int32),
        pltpu.VMEM((2, K), jnp.float32),
        pltpu.VMEM((2, K), jnp.int32),
        pltpu.VMEM((2, K, D), jnp.float32),
        pltpu.VMEM((2, K), jnp.int32),
        pltpu.VMEM((KTAIL,), jnp.int32),
        pltpu.VMEM((L,), jnp.int32),
        pltpu.VMEM_SHARED((N, D), jnp.float32),
        pltpu.SemaphoreType.DMA,
        pltpu.SemaphoreType.DMA,
        pltpu.SemaphoreType.DMA,
        pltpu.SemaphoreType.DMA,
        pltpu.SemaphoreType.DMA,
        pltpu.SemaphoreType.DMA,
    ],
)(_sc_edge_kernel)


def kernel(x, edge_index, edge_type, edge_norm, basis, root, bias):
    src = edge_index[0]
    dst = edge_index[1]

    xw = pl.pallas_call(
        _xw_body,
        grid=(N // BN,),
        in_specs=[
            pl.BlockSpec((BN, D), lambda i: (i, 0)),
            pl.BlockSpec((R, D, D), lambda i: (0, 0, 0)),
        ],
        out_specs=pl.BlockSpec((R, BN, D), lambda i: (0, i, 0)),
        out_shape=jax.ShapeDtypeStruct((R, N, D), jnp.float32),
    )(x, basis)
    xw_flat = xw.reshape(R * N, D)

    partial = _sc_edge(xw_flat, src, dst, edge_type, edge_norm)

    out = pl.pallas_call(
        _combine_body,
        grid=(N // BN,),
        in_specs=[
            pl.BlockSpec((NC, BN, D), lambda i: (0, i, 0)),
            pl.BlockSpec((BN, D), lambda i: (i, 0)),
            pl.BlockSpec((D, D), lambda i: (0, 0)),
            pl.BlockSpec((1, D), lambda i: (0, 0)),
        ],
        out_specs=pl.BlockSpec((BN, D), lambda i: (i, 0)),
        out_shape=jax.ShapeDtypeStruct((N, D), jnp.float32),
    )(partial, x, root, bias.reshape(1, D))
    return out


# speedup vs baseline: 30.9677x; 1.0012x over previous
import functools

import jax
import jax.numpy as jnp
from jax import lax
from jax.experimental import pallas as pl
from jax.experimental.pallas import tpu as pltpu
from jax.experimental.pallas import tpu_sc as plsc

N = 10000
E = 320000
D = 128
R = 8

NC = 2
NS = 16
L = 16
NW = NC * NS
EPW = E // NW
K = 128
NCHUNK = EPW // K
KTAIL = EPW - NCHUNK * K
RPT = 624
REXTRA = N - NS * RPT
BN = 1000


def _xw_body(x_ref, basis_ref, out_ref):
    xb = x_ref[...]
    for r in range(R):
        out_ref[r] = jnp.dot(xb, basis_ref[r],
                             preferred_element_type=jnp.float32)


def _combine_body(p_ref, x_ref, root_ref, bias_ref, out_ref):
    out_ref[...] = (p_ref[0] + p_ref[1] + bias_ref[...]
                    + jnp.dot(x_ref[...], root_ref[...],
                              preferred_element_type=jnp.float32))


def _sc_edge_kernel(xw_hbm, src_hbm, dst_hbm, typ_hbm, nrm_hbm, part_hbm,
                    srcv, typv, dstv, nrmv, gidxv, rowsf, dsts,
                    dstt, gidxt, acc, msem0, msem1, gsem0, gsem1,
                    ssem0, ssem1):
    cid = lax.axis_index("c")
    sid = lax.axis_index("s")
    wid = sid * NC + cid
    msem = (msem0, msem1)
    gsem = (gsem0, gsem1)
    ssem = (ssem0, ssem1)

    def zrow(i, carry):
        for g in range(D // L):
            rowsf[0, i, pl.ds(g * L, L)] = jnp.zeros((L,), jnp.float32)
        return carry
    lax.fori_loop(0, K, zrow, 0)
    r0 = sid * RPT
    nfull = RPT // K
    rem = RPT - nfull * K

    def zacc(j, carry):
        pltpu.sync_copy(rowsf.at[0], acc.at[pl.ds(r0 + j * K, K)])
        return carry
    lax.fori_loop(0, nfull, zacc, 0)
    if rem:
        pltpu.sync_copy(rowsf.at[0, pl.ds(0, rem)],
                        acc.at[pl.ds(r0 + nfull * K, rem)])

    @pl.when(sid == NS - 1)
    def _zero_tail():
        pltpu.sync_copy(rowsf.at[0, pl.ds(0, REXTRA)],
                        acc.at[pl.ds(NS * RPT, REXTRA)])
    plsc.subcore_barrier()

    def m_start(c, b):
        base = pl.multiple_of(wid * EPW + c * K, 8)
        pltpu.async_copy(src_hbm.at[pl.ds(base, K)], srcv.at[b], msem[b])
        pltpu.async_copy(dst_hbm.at[pl.ds(base, K)], dstv.at[b], msem[b])
        pltpu.async_copy(typ_hbm.at[pl.ds(base, K)], typv.at[b], msem[b])
        pltpu.async_copy(nrm_hbm.at[pl.ds(base, K)], nrmv.at[b], msem[b])

    def m_wait(b):
        pltpu.make_async_copy(src_hbm.at[pl.ds(0, K)], srcv.at[b],
                              msem[b]).wait()
        pltpu.make_async_copy(dst_hbm.at[pl.ds(0, K)], dstv.at[b],
                              msem[b]).wait()
        pltpu.make_async_copy(typ_hbm.at[pl.ds(0, K)], typv.at[b],
                              msem[b]).wait()
        pltpu.make_async_copy(nrm_hbm.at[pl.ds(0, K)], nrmv.at[b],
                              msem[b]).wait()

    def gidx_compute(b):
        for g in range(K // L):
            sl = pl.ds(g * L, L)
            gidxv[b, sl] = typv[b, sl] * N + srcv[b, sl]

    def g_start(b):
        pltpu.async_copy(xw_hbm.at[gidxv.at[b]], rowsf.at[b], gsem[b])

    def g_wait(b):
        pltpu.make_async_copy(xw_hbm.at[gidxv.at[b]], rowsf.at[b],
                              gsem[b]).wait()

    def _scale_edge(rowsf_e, nb):
        for h in range(D // L):
            sl = pl.ds(h * L, L)
            rowsf_e[sl] = rowsf_e[sl] * nb

    def scale(b):
        @plsc.parallel_loop(0, K, 1, unroll=8)
        def _body(e):
            nb = plsc.load_gather(nrmv.at[b],
                                  (jnp.full((L,), e, jnp.int32),))
            _scale_edge(rowsf.at[b, e], nb)

    def s_start(b):
        pltpu.async_copy(rowsf.at[b], acc.at[dsts.at[b]], ssem[b], add=True)

    def s_wait(b):
        pltpu.make_async_copy(rowsf.at[b], acc.at[dsts.at[b]],
                              ssem[b]).wait()

    def step(c, b):
        nb = 1 - b

        @pl.when(c + 1 < NCHUNK)
        def _advance():
            m_wait(nb)
            gidx_compute(nb)

            @pl.when(c >= 1)
            def _drain_prev_scatter():
                s_wait(nb)
            g_start(nb)
        g_wait(b)
        scale(b)
        for g in range(K // L):
            sl = pl.ds(g * L, L)
            dsts[b, sl] = dstv[b, sl]
        s_start(b)

        @pl.when(c + 2 < NCHUNK)
        def _prefetch():
            m_start(c + 2, b)

    m_start(0, 0)
    m_wait(0)
    gidx_compute(0)
    g_start(0)
    m_start(1, 1)

    def outer(i, carry):
        step(i * 2, 0)
        step(i * 2 + 1, 1)
        return carry
    lax.fori_loop(0, NCHUNK // 2, outer, 0)
    s_wait(1 - (NCHUNK % 2))
    s_wait(NCHUNK % 2)

    tbase = pl.multiple_of(wid * EPW + NCHUNK * K, 8)
    pltpu.sync_copy(src_hbm.at[pl.ds(tbase, KTAIL)],
                    srcv.at[0, pl.ds(0, KTAIL)])
    pltpu.sync_copy(typ_hbm.at[pl.ds(tbase, KTAIL)],
                    typv.at[0, pl.ds(0, KTAIL)])
    pltpu.sync_copy(dst_hbm.at[pl.ds(tbase, KTAIL)], dstt)
    pltpu.sync_copy(nrm_hbm.at[pl.ds(tbase, KTAIL)],
                    nrmv.at[0, pl.ds(0, KTAIL)])
    gidxt[...] = typv[0, pl.ds(0, L)] * N + srcv[0, pl.ds(0, L)]
    pltpu.async_copy(xw_hbm.at[gidxt], rowsf.at[0, pl.ds(0, KTAIL)],
                     gsem0).wait()

    def tail_scale(e, carry2):
        nb = plsc.load_gather(nrmv.at[0], (jnp.full((L,), e, jnp.int32),))
        _scale_edge(rowsf.at[0, e], nb)
        return carry2
    lax.fori_loop(0, KTAIL, tail_scale, 0)
    pltpu.sync_copy(rowsf.at[0, pl.ds(0, KTAIL)], acc.at[dstt], add=True)

    plsc.subcore_barrier()
    pltpu.sync_copy(acc.at[pl.ds(r0, RPT)], part_hbm.at[cid, pl.ds(r0, RPT)])

    @pl.when(sid == NS - 1)
    def _write_tail():
        pltpu.sync_copy(acc.at[pl.ds(NS * RPT, REXTRA)],
                        part_hbm.at[cid, pl.ds(NS * RPT, REXTRA)])


_sc_edge = functools.partial(
    pl.kernel,
    out_type=jax.ShapeDtypeStruct((NC, N, D), jnp.float32),
    mesh=plsc.VectorSubcoreMesh(core_axis_name="c", subcore_axis_name="s"),
    compiler_params=pltpu.CompilerParams(needs_layout_passes=False),
    scratch_types=[
        pltpu.VMEM((2, K), jnp.int32),
        pltpu.VMEM((2, K), jnp.int32),
        pltpu.VMEM((2, K), jnp.int32),
        pltpu.VMEM((2, K), jnp.float32),
        pltpu.VMEM((2, K), jnp.int32),
        pltpu.VMEM((2, K, D), jnp.float32),
        pltpu.VMEM((2, K), jnp.int32),
        pltpu.VMEM((KTAIL,), jnp.int32),
        pltpu.VMEM((L,), jnp.int32),
        pltpu.VMEM_SHARED((N, D), jnp.float32),
        pltpu.SemaphoreType.DMA,
        pltpu.SemaphoreType.DMA,
        pltpu.SemaphoreType.DMA,
        pltpu.SemaphoreType.DMA,
        pltpu.SemaphoreType.DMA,
        pltpu.SemaphoreType.DMA,
    ],
)(_sc_edge_kernel)


def kernel(x, edge_index, edge_type, edge_norm, basis, root, bias):
    xw = pl.pallas_call(
        _xw_body,
        grid=(N // BN,),
        in_specs=[
            pl.BlockSpec((BN, D), lambda i: (i, 0)),
            pl.BlockSpec((R, D, D), lambda i: (0, 0, 0)),
        ],
        out_specs=pl.BlockSpec((R, BN, D), lambda i: (0, i, 0)),
        out_shape=jax.ShapeDtypeStruct((R, N, D), jnp.float32),
    )(x, basis)

    partial = _sc_edge(xw.reshape(R * N, D), edge_index[0], edge_index[1],
                       edge_type, edge_norm)

    out = pl.pallas_call(
        _combine_body,
        grid=(N // BN,),
        in_specs=[
            pl.BlockSpec((NC, BN, D), lambda i: (0, i, 0)),
            pl.BlockSpec((BN, D), lambda i: (i, 0)),
            pl.BlockSpec((D, D), lambda i: (0, 0)),
            pl.BlockSpec((1, D), lambda i: (0, 0)),
        ],
        out_specs=pl.BlockSpec((BN, D), lambda i: (i, 0)),
        out_shape=jax.ShapeDtypeStruct((N, D), jnp.float32),
    )(partial, x, root, bias.reshape(1, D))
    return out
